# baseline (speedup 1.0000x reference)
import functools

import jax
import jax.numpy as jnp
from jax import lax
from jax.experimental import pallas as pl
from jax.experimental.pallas import tpu as pltpu
from jax.experimental.pallas import tpu_sc as plsc

B = 4096
S = 77
D = 768
SP = 80
RP = B * SP
L = 16

NC = 2
NS = 16
NW = NC * NS
ROWS_PER_W = RP // NW
C = 16
NCHUNK = ROWS_PER_W // C
NBUF = 4
PF = 2

_mesh = plsc.VectorSubcoreMesh(core_axis_name="c", subcore_axis_name="s")


@functools.partial(
    pl.kernel,
    out_type=jax.ShapeDtypeStruct((RP, D), jnp.float32),
    mesh=_mesh,
    scratch_types=[
        pltpu.VMEM((ROWS_PER_W,), jnp.int32),
        pltpu.VMEM((SP * D,), jnp.float32),
        pltpu.VMEM((NBUF, C, D), jnp.float32),
    ]
    + [pltpu.SemaphoreType.DMA] * (2 * NBUF),
)
def _embed(ids_hbm, tok_hbm, pos_hbm, out_hbm, idx_all, pos_v, work, *sems):
    gsem = sems[:NBUF]
    ssem = sems[NBUF:]
    wid = lax.axis_index("s") * NC + lax.axis_index("c")
    base = wid * ROWS_PER_W
    pltpu.sync_copy(pos_hbm, pos_v)
    pltpu.sync_copy(ids_hbm.at[pl.ds(base, ROWS_PER_W)], idx_all)

    def gather(i, b):
        return pltpu.make_async_copy(
            tok_hbm.at[idx_all.at[pl.ds(i * C, C)]], work.at[b], gsem[b])

    def scatter(i, b):
        return pltpu.make_async_copy(
            work.at[b], out_hbm.at[pl.ds(base + i * C, C)], ssem[b])

    for b in range(PF):
        gather(b, b).start()

    def group(it, carry):
        for b in range(NBUF):
            i = it * NBUF + b
            bg = (b + PF) % NBUF

            @pl.when(jnp.logical_and(i + PF < NCHUNK, i + PF >= NBUF))
            def _():
                scatter(i + PF - NBUF, bg).wait()

            @pl.when(i + PF < NCHUNK)
            def _():
                gather(i + PF, bg).start()

            gather(i, b).wait()
            p0 = C * lax.rem(i, SP // C)

            @plsc.parallel_loop(0, C)
            def row(j):
                @plsc.parallel_loop(0, D // L, unroll=8)
                def vec(v):
                    x = pos_v[pl.ds((p0 + j) * D + v * L, L)]
                    plsc.addupdate(work.at[b, j, pl.ds(v * L, L)], x)

            scatter(i, b).start()
        return carry

    lax.fori_loop(0, NCHUNK // NBUF, group, 0)

    for i in range(NCHUNK - NBUF, NCHUNK):
        scatter(i, i % NBUF).wait()


def kernel(inputs, token_table, position_table):
    ids = jnp.pad(inputs.astype(jnp.int32), ((0, 0), (0, SP - S))).reshape(RP)
    pos = jnp.pad(position_table, ((0, SP - S), (0, 0))).reshape(SP * D)
    out = _embed(ids, token_table, pos)
    return out.reshape(B, SP, D)[:, :S, :]
